# baseline (speedup 1.0000x reference)
import functools

import jax
import jax.numpy as jnp
from jax import lax
from jax.experimental import pallas as pl
from jax.experimental.pallas import tpu as pltpu
from jax.experimental.pallas import tpu_sc as plsc

_N = 10000
_E = 160000
_F_IN = 32
_EMB = 16
_NG = 64

_CHUNK = 128
_NW = 32
_CPW = 40
_ROUND = 20
_RROWS = _ROUND * _CHUNK
_SPAN = _CPW * _CHUNK
_E_PAD = _NW * _SPAN
_NCHP = _E_PAD // _CHUNK

_MSG_T = 2048


def _sc_mesh():
    return plsc.VectorSubcoreMesh(core_axis_name="c", subcore_axis_name="s")


@functools.lru_cache(maxsize=None)
def _make_gather(feat):
    round_ch = 10 if feat == _F_IN else _ROUND
    n_rounds = _CPW // round_ch
    rrows = round_ch * _CHUNK

    @functools.partial(
        pl.kernel,
        out_type=jax.ShapeDtypeStruct((_E_PAD, feat), jnp.float32),
        mesh=_sc_mesh(),
        scratch_types=[
            pltpu.VMEM((_CPW, _CHUNK), jnp.int32),
            pltpu.VMEM((rrows, feat), jnp.float32),
            pltpu.VMEM((rrows, feat), jnp.float32),
            pltpu.SemaphoreType.DMA,
            pltpu.SemaphoreType.DMA,
            pltpu.SemaphoreType.DMA,
        ],
        compiler_params=pltpu.CompilerParams(use_tc_tiling_on_sc=False),
    )
    def gather(table_hbm, idx_hbm, out_hbm, idx_v, rows_a, rows_b, sem_g,
               sem_w0, sem_w1):
        wid = lax.axis_index("s") * 2 + lax.axis_index("c")
        cbase = wid * _CPW
        ebase = wid * _SPAN

        pltpu.sync_copy(idx_hbm.at[pl.ds(cbase, _CPW)], idx_v)

        def fires(buf, joff):
            def body(j, carry):
                pltpu.async_copy(
                    table_hbm.at[idx_v.at[joff + j]],
                    buf.at[pl.ds(j * _CHUNK, _CHUNK)], sem_g)
                return carry
            lax.fori_loop(0, round_ch, body, 0)

        def drains(buf, joff):
            def body(j, carry):
                pltpu.make_async_copy(
                    table_hbm.at[idx_v.at[joff + j]],
                    buf.at[pl.ds(j * _CHUNK, _CHUNK)], sem_g).wait()
                return carry
            lax.fori_loop(0, round_ch, body, 0)

        bufs = (rows_a, rows_b)
        wsems = (sem_w0, sem_w1)
        pending = [None, None]
        for r in range(n_rounds):
            slot = r % 2
            if pending[slot] is not None:
                pending[slot].wait()
            joff = r * round_ch
            fires(bufs[slot], joff)
            drains(bufs[slot], joff)
            pending[slot] = pltpu.async_copy(
                bufs[slot],
                out_hbm.at[pl.ds(ebase + joff * _CHUNK, rrows)], wsems[slot])
        for d in pending:
            if d is not None:
                d.wait()

    return gather


@functools.lru_cache(maxsize=None)
def _make_scatter_add():
    @functools.partial(
        pl.kernel,
        out_type=jax.ShapeDtypeStruct((2, _N, _EMB), jnp.float32),
        mesh=_sc_mesh(),
        scratch_types=[
            pltpu.VMEM((_CPW, _CHUNK), jnp.int32),
            pltpu.VMEM((_RROWS, _EMB), jnp.float32),
            pltpu.VMEM((_RROWS, _EMB), jnp.float32),
            pltpu.VMEM_SHARED((_N, _EMB), jnp.float32),
            pltpu.SemaphoreType.DMA,
        ],
        compiler_params=pltpu.CompilerParams(use_tc_tiling_on_sc=False),
    )
    def scatter_add(msg_hbm, idx_hbm, zeros_hbm, out_hbm, idx_v, msg_a,
                    msg_b, acc_sh, sem_s):
        c = lax.axis_index("c")
        s = lax.axis_index("s")
        wid = s * 2 + c
        cbase = wid * _CPW
        ebase = wid * _SPAN

        pltpu.sync_copy(zeros_hbm.at[pl.ds(s * 625, 625)],
                        acc_sh.at[pl.ds(s * 625, 625)])
        plsc.subcore_barrier()

        pltpu.sync_copy(idx_hbm.at[pl.ds(cbase, _CPW)], idx_v)
        pltpu.sync_copy(msg_hbm.at[pl.ds(ebase, _RROWS)], msg_a)

        def fires(buf, joff):
            def body(j, carry):
                pltpu.async_copy(
                    buf.at[pl.ds(j * _CHUNK, _CHUNK)],
                    acc_sh.at[idx_v.at[joff + j]], sem_s, add=True)
                return carry
            lax.fori_loop(0, _ROUND, body, 0)

        def drains(buf, joff):
            def body(j, carry):
                pltpu.make_async_copy(
                    buf.at[pl.ds(j * _CHUNK, _CHUNK)],
                    acc_sh.at[idx_v.at[joff + j]], sem_s).wait()
                return carry
            lax.fori_loop(0, _ROUND, body, 0)

        fires(msg_a, 0)
        pltpu.sync_copy(msg_hbm.at[pl.ds(ebase + _RROWS, _RROWS)], msg_b)
        drains(msg_a, 0)
        fires(msg_b, _ROUND)
        drains(msg_b, _ROUND)

        plsc.subcore_barrier()
        pltpu.sync_copy(acc_sh.at[pl.ds(s * 625, 625)],
                        out_hbm.at[c].at[pl.ds(s * 625, 625)])

    return scatter_add


def _make_msg_body(feat):
    def body(ea_ref, xg_ref, w1_ref, b1_ref, w2_ref, b2_ref, out_ref):
        h = jnp.maximum(
            lax.dot(ea_ref[...], w1_ref[...],
                    preferred_element_type=jnp.float32) + b1_ref[...], 0.0)
        z = lax.dot(h, w2_ref[...],
                    preferred_element_type=jnp.float32) + b2_ref[...]
        z = z.astype(jnp.bfloat16).astype(jnp.float32)
        xg = xg_ref[...].astype(jnp.bfloat16).astype(jnp.float32)
        p = jnp.concatenate(
            [xg[:, i:i + 1] * z[:, i * _EMB:(i + 1) * _EMB]
             for i in range(feat)], axis=1)
        width = feat * _EMB
        while width > _EMB:
            width //= 2
            p = p[:, :width] + p[:, width:2 * width]
        row = (pl.program_id(0) * _MSG_T
               + lax.broadcasted_iota(jnp.int32, (_MSG_T, 1), 0))
        out_ref[...] = jnp.where(row < _E, p, 0.0)
    return body


def _msg_call(ea_pad, xg, w1, b1, w2, b2):
    feat = xg.shape[1]
    full = lambda shape: pl.BlockSpec(shape, lambda i: (0, 0))
    return pl.pallas_call(
        _make_msg_body(feat),
        grid=(_E_PAD // _MSG_T,),
        in_specs=[
            pl.BlockSpec((_MSG_T, 16), lambda i: (i, 0)),
            pl.BlockSpec((_MSG_T, feat), lambda i: (i, 0)),
            full((16, 16)),
            full((1, 16)),
            full((16, feat * _EMB)),
            full((1, feat * _EMB)),
        ],
        out_specs=pl.BlockSpec((_MSG_T, _EMB), lambda i: (i, 0)),
        out_shape=jax.ShapeDtypeStruct((_E_PAD, _EMB), jnp.float32),
    )(ea_pad, xg, w1, b1.reshape(1, 16), w2, b2.reshape(1, feat * _EMB))


def _update_body(p0_ref, p1_ref, x_ref, root_ref, bias_ref, out_ref):
    agg = p0_ref[...] + p1_ref[...]
    out_ref[...] = jnp.maximum(
        agg + lax.dot(x_ref[...], root_ref[...],
                      preferred_element_type=jnp.float32) + bias_ref[...], 0.0)


def _update_call(p0, p1, x, root, bias):
    feat = x.shape[1]
    tile = 1000
    full = lambda shape: pl.BlockSpec(shape, lambda i: (0, 0))
    return pl.pallas_call(
        _update_body,
        grid=(_N // tile,),
        in_specs=[
            pl.BlockSpec((tile, _EMB), lambda i: (i, 0)),
            pl.BlockSpec((tile, _EMB), lambda i: (i, 0)),
            pl.BlockSpec((tile, feat), lambda i: (i, 0)),
            full((feat, _EMB)),
            full((1, _EMB)),
        ],
        out_specs=pl.BlockSpec((tile, _EMB), lambda i: (i, 0)),
        out_shape=jax.ShapeDtypeStruct((_N, _EMB), jnp.float32),
    )(p0, p1, x, root, bias.reshape(1, _EMB))


def _final_body(p0_ref, p1_ref, x_ref, root_ref, bias_ref, batch_ref,
                l0w_ref, l0b_ref, l1w_ref, l1b_ref, out_ref, pool_ref):
    x2 = jnp.maximum(
        p0_ref[...] + p1_ref[...]
        + lax.dot(x_ref[...], root_ref[...],
                  preferred_element_type=jnp.float32) + bias_ref[...], 0.0)
    batch = batch_ref[...]

    def body(g, carry):
        m = jnp.where(batch == g, x2, -jnp.inf)
        pool_ref[pl.ds(g, 1), :] = jnp.max(m, axis=0, keepdims=True)
        return carry

    lax.fori_loop(0, _NG, body, 0)
    hidden = lax.dot(pool_ref[...], l0w_ref[...],
                     preferred_element_type=jnp.float32) + l0b_ref[...]
    out_ref[...] = lax.dot(hidden, l1w_ref[...],
                           preferred_element_type=jnp.float32) + l1b_ref[...]


def _final_call(p0, p1, x1, root, bias, batch, l0w, l0b, l1w, l1b):
    return pl.pallas_call(
        _final_body,
        out_shape=jax.ShapeDtypeStruct((_NG, 1), jnp.float32),
        scratch_shapes=[pltpu.VMEM((_NG, _EMB), jnp.float32)],
    )(p0, p1, x1, root, bias.reshape(1, _EMB), batch.reshape(_N, 1),
      l0w, l0b.reshape(1, _EMB), l1w, l1b.reshape(1, 1))


def _pad_idx(idx):
    return jnp.concatenate(
        [idx, jnp.zeros((_E_PAD - _E,), jnp.int32)]).reshape(_NCHP, _CHUNK)


@jax.jit
def kernel(x_p, x_d, edge_attr_p, edge_attr_d, edge_index_p, x_p_batch,
           nn0_W1, nn0_b1, nn0_W2, nn0_b2,
           nn1_W1, nn1_b1, nn1_W2, nn1_b2,
           root0, bias0, root1, bias1,
           lin0_W, lin0_b, lin1_W, lin1_b):
    src2d = _pad_idx(edge_index_p[0])
    dst2d = _pad_idx(edge_index_p[1])
    ea_pad = jnp.concatenate(
        [edge_attr_p, jnp.zeros((_E_PAD - _E, 16), jnp.float32)])

    zeros_n = jnp.zeros((_N, _EMB), jnp.float32)

    scatter_add = _make_scatter_add()

    xg0 = _make_gather(_F_IN)(x_p, src2d)
    msg0 = _msg_call(ea_pad, xg0, nn0_W1, nn0_b1, nn0_W2, nn0_b2)
    parts0 = scatter_add(msg0, dst2d, zeros_n)
    x1 = _update_call(parts0[0], parts0[1], x_p, root0, bias0)

    xg1 = _make_gather(_EMB)(x1, src2d)
    msg1 = _msg_call(ea_pad, xg1, nn1_W1, nn1_b1, nn1_W2, nn1_b2)
    parts1 = scatter_add(msg1, dst2d, zeros_n)

    return _final_call(parts1[0], parts1[1], x1, root1, bias1, x_p_batch,
                       lin0_W, lin0_b, lin1_W, lin1_b)


# speedup vs baseline: 1.5964x; 1.5964x over previous
import functools

import jax
import jax.numpy as jnp
from jax import lax
from jax.experimental import pallas as pl
from jax.experimental.pallas import tpu as pltpu
from jax.experimental.pallas import tpu_sc as plsc

_N = 10000
_E = 160000
_F_IN = 32
_EMB = 16
_NG = 64

_CHUNK = 128
_NW = 32
_CPW = 40
_ROUND = 20
_RROWS = _ROUND * _CHUNK
_SPAN = _CPW * _CHUNK
_E_PAD = _NW * _SPAN
_NCHP = _E_PAD // _CHUNK

_MSG_T = 2048


def _sc_mesh():
    return plsc.VectorSubcoreMesh(core_axis_name="c", subcore_axis_name="s")


@functools.lru_cache(maxsize=None)
def _make_gather(feat):
    round_ch = 10 if feat == _F_IN else _ROUND
    n_rounds = _CPW // round_ch
    rrows = round_ch * _CHUNK

    @functools.partial(
        pl.kernel,
        out_type=jax.ShapeDtypeStruct((_E_PAD, feat), jnp.float32),
        mesh=_sc_mesh(),
        scratch_types=[
            pltpu.VMEM((_CPW, _CHUNK), jnp.int32),
            pltpu.VMEM((rrows, feat), jnp.float32),
            pltpu.VMEM((rrows, feat), jnp.float32),
            pltpu.SemaphoreType.DMA,
            pltpu.SemaphoreType.DMA,
            pltpu.SemaphoreType.DMA,
        ],
        compiler_params=pltpu.CompilerParams(use_tc_tiling_on_sc=False),
    )
    def gather(table_hbm, idx_hbm, out_hbm, idx_v, rows_a, rows_b, sem_g,
               sem_w0, sem_w1):
        wid = lax.axis_index("s") * 2 + lax.axis_index("c")
        cbase = wid * _CPW
        ebase = wid * _SPAN

        pltpu.sync_copy(idx_hbm.at[pl.ds(cbase, _CPW)], idx_v)

        def fires(buf, joff):
            def body(j, carry):
                pltpu.async_copy(
                    table_hbm.at[idx_v.at[joff + j]],
                    buf.at[pl.ds(j * _CHUNK, _CHUNK)], sem_g)
                return carry
            lax.fori_loop(0, round_ch, body, 0)

        def drains(buf, joff):
            def body(j, carry):
                pltpu.make_async_copy(
                    table_hbm.at[idx_v.at[joff + j]],
                    buf.at[pl.ds(j * _CHUNK, _CHUNK)], sem_g).wait()
                return carry
            lax.fori_loop(0, round_ch, body, 0)

        bufs = (rows_a, rows_b)
        wsems = (sem_w0, sem_w1)
        pending = [None, None]
        for r in range(n_rounds):
            slot = r % 2
            if pending[slot] is not None:
                pending[slot].wait()
            joff = r * round_ch
            fires(bufs[slot], joff)
            drains(bufs[slot], joff)
            pending[slot] = pltpu.async_copy(
                bufs[slot],
                out_hbm.at[pl.ds(ebase + joff * _CHUNK, rrows)], wsems[slot])
        for d in pending:
            if d is not None:
                d.wait()

    return gather


@functools.lru_cache(maxsize=None)
def _make_scatter_add():
    @functools.partial(
        pl.kernel,
        out_type=jax.ShapeDtypeStruct((2, _N, _EMB), jnp.float32),
        mesh=_sc_mesh(),
        scratch_types=[
            pltpu.VMEM((_CPW, _CHUNK), jnp.int32),
            pltpu.VMEM((_RROWS, _EMB), jnp.float32),
            pltpu.VMEM((_RROWS, _EMB), jnp.float32),
            pltpu.VMEM_SHARED((_N, _EMB), jnp.float32),
            pltpu.SemaphoreType.DMA,
        ],
        compiler_params=pltpu.CompilerParams(use_tc_tiling_on_sc=False),
    )
    def scatter_add(msg_hbm, idx_hbm, zeros_hbm, out_hbm, idx_v, msg_a,
                    msg_b, acc_sh, sem_s):
        c = lax.axis_index("c")
        s = lax.axis_index("s")
        wid = s * 2 + c
        cbase = wid * _CPW
        ebase = wid * _SPAN

        pltpu.sync_copy(zeros_hbm.at[pl.ds(s * 625, 625)],
                        acc_sh.at[pl.ds(s * 625, 625)])
        plsc.subcore_barrier()

        pltpu.sync_copy(idx_hbm.at[pl.ds(cbase, _CPW)], idx_v)
        pltpu.sync_copy(msg_hbm.at[pl.ds(ebase, _RROWS)], msg_a)

        def fires(buf, joff):
            def body(j, carry):
                pltpu.async_copy(
                    buf.at[pl.ds(j * _CHUNK, _CHUNK)],
                    acc_sh.at[idx_v.at[joff + j]], sem_s, add=True)
                return carry
            lax.fori_loop(0, _ROUND, body, 0)

        def drains(buf, joff):
            def body(j, carry):
                pltpu.make_async_copy(
                    buf.at[pl.ds(j * _CHUNK, _CHUNK)],
                    acc_sh.at[idx_v.at[joff + j]], sem_s).wait()
                return carry
            lax.fori_loop(0, _ROUND, body, 0)

        fires(msg_a, 0)
        pltpu.sync_copy(msg_hbm.at[pl.ds(ebase + _RROWS, _RROWS)], msg_b)
        drains(msg_a, 0)
        fires(msg_b, _ROUND)
        drains(msg_b, _ROUND)

        plsc.subcore_barrier()
        pltpu.sync_copy(acc_sh.at[pl.ds(s * 625, 625)],
                        out_hbm.at[c].at[pl.ds(s * 625, 625)])

    return scatter_add


def _make_msg_body(feat):
    def body(ea_ref, xg_ref, w1_ref, b1_ref, w2_ref, b2_ref, r_ref, out_ref):
        h = jnp.maximum(
            lax.dot(ea_ref[...], w1_ref[...],
                    preferred_element_type=jnp.float32) + b1_ref[...], 0.0)
        z = lax.dot(h, w2_ref[...],
                    preferred_element_type=jnp.float32) + b2_ref[...]
        z = z.astype(jnp.bfloat16).astype(jnp.float32)
        xg = xg_ref[...].astype(jnp.bfloat16).astype(jnp.float32)
        xb = lax.dot(xg, r_ref[...], preferred_element_type=jnp.float32)
        p = z * xb
        width = feat * _EMB
        while width > _EMB:
            width //= 2
            p = p[:, :width] + p[:, width:2 * width]
        row = (pl.program_id(0) * _MSG_T
               + lax.broadcasted_iota(jnp.int32, (_MSG_T, 1), 0))
        out_ref[...] = jnp.where(row < _E, p, 0.0)
    return body


def _msg_call(ea_pad, xg, w1, b1, w2, b2, r_mat):
    feat = xg.shape[1]
    full = lambda shape: pl.BlockSpec(shape, lambda i: (0, 0))
    return pl.pallas_call(
        _make_msg_body(feat),
        grid=(_E_PAD // _MSG_T,),
        in_specs=[
            pl.BlockSpec((_MSG_T, 16), lambda i: (i, 0)),
            pl.BlockSpec((_MSG_T, feat), lambda i: (i, 0)),
            full((16, 16)),
            full((1, 16)),
            full((16, feat * _EMB)),
            full((1, feat * _EMB)),
            full((feat, feat * _EMB)),
        ],
        out_specs=pl.BlockSpec((_MSG_T, _EMB), lambda i: (i, 0)),
        out_shape=jax.ShapeDtypeStruct((_E_PAD, _EMB), jnp.float32),
    )(ea_pad, xg, w1, b1.reshape(1, 16), w2, b2.reshape(1, feat * _EMB),
      r_mat)


def _update_body(p0_ref, p1_ref, x_ref, root_ref, bias_ref, out_ref):
    agg = p0_ref[...] + p1_ref[...]
    out_ref[...] = jnp.maximum(
        agg + lax.dot(x_ref[...], root_ref[...],
                      preferred_element_type=jnp.float32) + bias_ref[...], 0.0)


def _update_call(p0, p1, x, root, bias):
    feat = x.shape[1]
    tile = 1000
    full = lambda shape: pl.BlockSpec(shape, lambda i: (0, 0))
    return pl.pallas_call(
        _update_body,
        grid=(_N // tile,),
        in_specs=[
            pl.BlockSpec((tile, _EMB), lambda i: (i, 0)),
            pl.BlockSpec((tile, _EMB), lambda i: (i, 0)),
            pl.BlockSpec((tile, feat), lambda i: (i, 0)),
            full((feat, _EMB)),
            full((1, _EMB)),
        ],
        out_specs=pl.BlockSpec((tile, _EMB), lambda i: (i, 0)),
        out_shape=jax.ShapeDtypeStruct((_N, _EMB), jnp.float32),
    )(p0, p1, x, root, bias.reshape(1, _EMB))


def _final_body(p0_ref, p1_ref, x_ref, root_ref, bias_ref, batch_ref,
                l0w_ref, l0b_ref, l1w_ref, l1b_ref, out_ref, pool_ref):
    x2 = jnp.maximum(
        p0_ref[...] + p1_ref[...]
        + lax.dot(x_ref[...], root_ref[...],
                  preferred_element_type=jnp.float32) + bias_ref[...], 0.0)
    batch = batch_ref[...]

    def body(g, carry):
        m = jnp.where(batch == g, x2, -jnp.inf)
        pool_ref[pl.ds(g, 1), :] = jnp.max(m, axis=0, keepdims=True)
        return carry

    lax.fori_loop(0, _NG, body, 0)
    hidden = lax.dot(pool_ref[...], l0w_ref[...],
                     preferred_element_type=jnp.float32) + l0b_ref[...]
    out_ref[...] = lax.dot(hidden, l1w_ref[...],
                           preferred_element_type=jnp.float32) + l1b_ref[...]


def _final_call(p0, p1, x1, root, bias, batch, l0w, l0b, l1w, l1b):
    return pl.pallas_call(
        _final_body,
        out_shape=jax.ShapeDtypeStruct((_NG, 1), jnp.float32),
        scratch_shapes=[pltpu.VMEM((_NG, _EMB), jnp.float32)],
    )(p0, p1, x1, root, bias.reshape(1, _EMB), batch.reshape(_N, 1),
      l0w, l0b.reshape(1, _EMB), l1w, l1b.reshape(1, 1))


def _pad_idx(idx):
    return jnp.concatenate(
        [idx, jnp.zeros((_E_PAD - _E,), jnp.int32)]).reshape(_NCHP, _CHUNK)


@jax.jit
def kernel(x_p, x_d, edge_attr_p, edge_attr_d, edge_index_p, x_p_batch,
           nn0_W1, nn0_b1, nn0_W2, nn0_b2,
           nn1_W1, nn1_b1, nn1_W2, nn1_b2,
           root0, bias0, root1, bias1,
           lin0_W, lin0_b, lin1_W, lin1_b):
    src2d = _pad_idx(edge_index_p[0])
    dst2d = _pad_idx(edge_index_p[1])
    ea_pad = jnp.concatenate(
        [edge_attr_p, jnp.zeros((_E_PAD - _E, 16), jnp.float32)])

    zeros_n = jnp.zeros((_N, _EMB), jnp.float32)

    def bcast_mat(feat):
        cols = jnp.arange(feat * _EMB) // _EMB
        return (cols[None, :] == jnp.arange(feat)[:, None]).astype(jnp.float32)

    r32 = bcast_mat(_F_IN)
    r16 = bcast_mat(_EMB)

    scatter_add = _make_scatter_add()

    xg0 = _make_gather(_F_IN)(x_p, src2d)
    msg0 = _msg_call(ea_pad, xg0, nn0_W1, nn0_b1, nn0_W2, nn0_b2, r32)
    parts0 = scatter_add(msg0, dst2d, zeros_n)
    x1 = _update_call(parts0[0], parts0[1], x_p, root0, bias0)

    xg1 = _make_gather(_EMB)(x1, src2d)
    msg1 = _msg_call(ea_pad, xg1, nn1_W1, nn1_b1, nn1_W2, nn1_b2, r16)
    parts1 = scatter_add(msg1, dst2d, zeros_n)

    return _final_call(parts1[0], parts1[1], x1, root1, bias1, x_p_batch,
                       lin0_W, lin0_b, lin1_W, lin1_b)


# speedup vs baseline: 1.6708x; 1.0466x over previous
import functools

import jax
import jax.numpy as jnp
from jax import lax
from jax.experimental import pallas as pl
from jax.experimental.pallas import tpu as pltpu
from jax.experimental.pallas import tpu_sc as plsc

_N = 10000
_E = 160000
_F_IN = 32
_EMB = 16
_NG = 64

_CHUNK = 128
_NW = 32
_CPW = 40
_ROUND = 20
_RROWS = _ROUND * _CHUNK
_SPAN = _CPW * _CHUNK
_E_PAD = _NW * _SPAN
_NCHP = _E_PAD // _CHUNK

_MSG_T = 2048


def _sc_mesh():
    return plsc.VectorSubcoreMesh(core_axis_name="c", subcore_axis_name="s")


@functools.lru_cache(maxsize=None)
def _make_gather(feat):
    round_ch = 10 if feat == _F_IN else _ROUND
    n_rounds = _CPW // round_ch
    rrows = round_ch * _CHUNK

    @functools.partial(
        pl.kernel,
        out_type=jax.ShapeDtypeStruct((_E_PAD, feat), jnp.float32),
        mesh=_sc_mesh(),
        scratch_types=[
            pltpu.VMEM((_CPW, _CHUNK), jnp.int32),
            pltpu.VMEM((rrows, feat), jnp.float32),
            pltpu.VMEM((rrows, feat), jnp.float32),
            pltpu.SemaphoreType.DMA,
            pltpu.SemaphoreType.DMA,
            pltpu.SemaphoreType.DMA,
        ],
        compiler_params=pltpu.CompilerParams(use_tc_tiling_on_sc=False),
    )
    def gather(table_hbm, idx_hbm, out_hbm, idx_v, rows_a, rows_b, sem_g,
               sem_w0, sem_w1):
        wid = lax.axis_index("s") * 2 + lax.axis_index("c")
        cbase = wid * _CPW
        ebase = wid * _SPAN

        pltpu.sync_copy(idx_hbm.at[pl.ds(cbase, _CPW)], idx_v)

        def fires(buf, joff):
            def body(j, carry):
                pltpu.async_copy(
                    table_hbm.at[idx_v.at[joff + j]],
                    buf.at[pl.ds(j * _CHUNK, _CHUNK)], sem_g)
                return carry
            lax.fori_loop(0, round_ch, body, 0)

        def drains(buf, joff):
            def body(j, carry):
                pltpu.make_async_copy(
                    table_hbm.at[idx_v.at[joff + j]],
                    buf.at[pl.ds(j * _CHUNK, _CHUNK)], sem_g).wait()
                return carry
            lax.fori_loop(0, round_ch, body, 0)

        bufs = (rows_a, rows_b)
        wsems = (sem_w0, sem_w1)
        pending = [None, None]
        for r in range(n_rounds):
            slot = r % 2
            if pending[slot] is not None:
                pending[slot].wait()
            joff = r * round_ch
            fires(bufs[slot], joff)
            drains(bufs[slot], joff)
            pending[slot] = pltpu.async_copy(
                bufs[slot],
                out_hbm.at[pl.ds(ebase + joff * _CHUNK, rrows)], wsems[slot])
        for d in pending:
            if d is not None:
                d.wait()

    return gather


_MID_R = 10
_MID_RROWS = _MID_R * _CHUNK
_CPT = _NCHP // 16


@functools.lru_cache(maxsize=None)
def _make_fused_mid():
    @functools.partial(
        pl.kernel,
        out_type=(jax.ShapeDtypeStruct((_E_PAD, _EMB), jnp.float32),
                  jax.ShapeDtypeStruct((_N, _EMB), jnp.float32)),
        mesh=_sc_mesh(),
        scratch_types=[
            pltpu.VMEM((_CPT, _CHUNK), jnp.int32),
            pltpu.VMEM((_MID_RROWS, _EMB), jnp.float32),
            pltpu.VMEM((_MID_RROWS, _EMB), jnp.float32),
            pltpu.VMEM((625, _EMB), jnp.float32),
            pltpu.VMEM((625, _EMB), jnp.float32),
            pltpu.VMEM_SHARED((_N, _EMB), jnp.float32),
            pltpu.SemaphoreType.DMA,
            pltpu.SemaphoreType.DMA,
            pltpu.SemaphoreType.DMA,
        ],
        compiler_params=pltpu.CompilerParams(use_tc_tiling_on_sc=False),
    )
    def fused_mid(msg_hbm, dst_hbm, src_hbm, r0_hbm, zeros_hbm,
                  xg1_hbm, x1_hbm, idx_v, buf_a, buf_b, row_v, r0_v, acc_sh,
                  sem_s, sem_w0, sem_w1):
        c = lax.axis_index("c")
        s = lax.axis_index("s")

        pltpu.sync_copy(zeros_hbm.at[pl.ds(s * 625, 625)],
                        acc_sh.at[pl.ds(s * 625, 625)])
        plsc.subcore_barrier()

        pltpu.sync_copy(dst_hbm.at[pl.ds(s * _CPT, _CPT)], idx_v)

        def sfires(buf, joff):
            def body(j, carry):
                pltpu.async_copy(
                    buf.at[pl.ds(j * _CHUNK, _CHUNK)],
                    acc_sh.at[idx_v.at[joff + j]], sem_s, add=True)
                return carry
            lax.fori_loop(0, _MID_R, body, 0)

        def sdrains(buf, joff):
            def body(j, carry):
                pltpu.make_async_copy(
                    buf.at[pl.ds(j * _CHUNK, _CHUNK)],
                    acc_sh.at[idx_v.at[joff + j]], sem_s).wait()
                return carry
            lax.fori_loop(0, _MID_R, body, 0)

        ebase = s * _CPT * _CHUNK
        n_rounds = _CPT // _MID_R
        pltpu.sync_copy(msg_hbm.at[pl.ds(ebase, _MID_RROWS)], buf_a)
        for r in range(n_rounds):
            buf = buf_a if r % 2 == 0 else buf_b
            nxt = buf_b if r % 2 == 0 else buf_a
            sfires(buf, r * _MID_R)
            if r + 1 < n_rounds:
                pltpu.sync_copy(
                    msg_hbm.at[pl.ds(ebase + (r + 1) * _MID_RROWS,
                                     _MID_RROWS)], nxt)
            sdrains(buf, r * _MID_R)
        plsc.subcore_barrier()

        pltpu.sync_copy(acc_sh.at[pl.ds(s * 625, 625)], row_v)
        pltpu.sync_copy(r0_hbm.at[pl.ds(s * 625, 625)], r0_v)

        def urow(i, carry):
            row_v[i] = jnp.maximum(row_v[i] + r0_v[i], 0.0)
            return carry
        lax.fori_loop(0, 625, urow, 0)

        pltpu.sync_copy(row_v, acc_sh.at[pl.ds(s * 625, 625)])

        @pl.when(c == 0)
        def _():
            pltpu.sync_copy(row_v, x1_hbm.at[pl.ds(s * 625, 625)])
        plsc.subcore_barrier()

        wid = s * 2 + c
        pltpu.sync_copy(src_hbm.at[pl.ds(wid * _CPW, _CPW)],
                        idx_v.at[pl.ds(0, _CPW)])

        def gfires(buf, joff):
            def body(j, carry):
                pltpu.async_copy(
                    acc_sh.at[idx_v.at[joff + j]],
                    buf.at[pl.ds(j * _CHUNK, _CHUNK)], sem_s)
                return carry
            lax.fori_loop(0, _MID_R, body, 0)

        def gdrains(buf, joff):
            def body(j, carry):
                pltpu.make_async_copy(
                    acc_sh.at[idx_v.at[joff + j]],
                    buf.at[pl.ds(j * _CHUNK, _CHUNK)], sem_s).wait()
                return carry
            lax.fori_loop(0, _MID_R, body, 0)

        gbase = wid * _SPAN
        wsems = (sem_w0, sem_w1)
        pending = [None, None]
        for r in range(_CPW // _MID_R):
            slot = r % 2
            buf = buf_a if slot == 0 else buf_b
            if pending[slot] is not None:
                pending[slot].wait()
            gfires(buf, r * _MID_R)
            gdrains(buf, r * _MID_R)
            pending[slot] = pltpu.async_copy(
                buf, xg1_hbm.at[pl.ds(gbase + r * _MID_RROWS, _MID_RROWS)],
                wsems[slot])
        for d in pending:
            if d is not None:
                d.wait()

    return fused_mid


@functools.lru_cache(maxsize=None)
def _make_scatter_add():
    @functools.partial(
        pl.kernel,
        out_type=jax.ShapeDtypeStruct((2, _N, _EMB), jnp.float32),
        mesh=_sc_mesh(),
        scratch_types=[
            pltpu.VMEM((_CPW, _CHUNK), jnp.int32),
            pltpu.VMEM((_RROWS, _EMB), jnp.float32),
            pltpu.VMEM((_RROWS, _EMB), jnp.float32),
            pltpu.VMEM_SHARED((_N, _EMB), jnp.float32),
            pltpu.SemaphoreType.DMA,
        ],
        compiler_params=pltpu.CompilerParams(use_tc_tiling_on_sc=False),
    )
    def scatter_add(msg_hbm, idx_hbm, zeros_hbm, out_hbm, idx_v, msg_a,
                    msg_b, acc_sh, sem_s):
        c = lax.axis_index("c")
        s = lax.axis_index("s")
        wid = s * 2 + c
        cbase = wid * _CPW
        ebase = wid * _SPAN

        pltpu.sync_copy(zeros_hbm.at[pl.ds(s * 625, 625)],
                        acc_sh.at[pl.ds(s * 625, 625)])
        plsc.subcore_barrier()

        pltpu.sync_copy(idx_hbm.at[pl.ds(cbase, _CPW)], idx_v)
        pltpu.sync_copy(msg_hbm.at[pl.ds(ebase, _RROWS)], msg_a)

        def fires(buf, joff):
            def body(j, carry):
                pltpu.async_copy(
                    buf.at[pl.ds(j * _CHUNK, _CHUNK)],
                    acc_sh.at[idx_v.at[joff + j]], sem_s, add=True)
                return carry
            lax.fori_loop(0, _ROUND, body, 0)

        def drains(buf, joff):
            def body(j, carry):
                pltpu.make_async_copy(
                    buf.at[pl.ds(j * _CHUNK, _CHUNK)],
                    acc_sh.at[idx_v.at[joff + j]], sem_s).wait()
                return carry
            lax.fori_loop(0, _ROUND, body, 0)

        fires(msg_a, 0)
        pltpu.sync_copy(msg_hbm.at[pl.ds(ebase + _RROWS, _RROWS)], msg_b)
        drains(msg_a, 0)
        fires(msg_b, _ROUND)
        drains(msg_b, _ROUND)

        plsc.subcore_barrier()
        pltpu.sync_copy(acc_sh.at[pl.ds(s * 625, 625)],
                        out_hbm.at[c].at[pl.ds(s * 625, 625)])

    return scatter_add


def _make_msg_body(feat):
    def body(ea_ref, xg_ref, w1_ref, b1_ref, w2_ref, b2_ref, r_ref, out_ref):
        h = jnp.maximum(
            lax.dot(ea_ref[...], w1_ref[...],
                    preferred_element_type=jnp.float32) + b1_ref[...], 0.0)
        z = lax.dot(h, w2_ref[...],
                    preferred_element_type=jnp.float32) + b2_ref[...]
        z = z.astype(jnp.bfloat16).astype(jnp.float32)
        xg = xg_ref[...].astype(jnp.bfloat16).astype(jnp.float32)
        xb = lax.dot(xg, r_ref[...], preferred_element_type=jnp.float32)
        p = z * xb
        width = feat * _EMB
        while width > _EMB:
            width //= 2
            p = p[:, :width] + p[:, width:2 * width]
        row = (pl.program_id(0) * _MSG_T
               + lax.broadcasted_iota(jnp.int32, (_MSG_T, 1), 0))
        out_ref[...] = jnp.where(row < _E, p, 0.0)
    return body


def _msg_call(ea_pad, xg, w1, b1, w2, b2, r_mat):
    feat = xg.shape[1]
    full = lambda shape: pl.BlockSpec(shape, lambda i: (0, 0))
    return pl.pallas_call(
        _make_msg_body(feat),
        grid=(_E_PAD // _MSG_T,),
        in_specs=[
            pl.BlockSpec((_MSG_T, 16), lambda i: (i, 0)),
            pl.BlockSpec((_MSG_T, feat), lambda i: (i, 0)),
            full((16, 16)),
            full((1, 16)),
            full((16, feat * _EMB)),
            full((1, feat * _EMB)),
            full((feat, feat * _EMB)),
        ],
        out_specs=pl.BlockSpec((_MSG_T, _EMB), lambda i: (i, 0)),
        out_shape=jax.ShapeDtypeStruct((_E_PAD, _EMB), jnp.float32),
    )(ea_pad, xg, w1, b1.reshape(1, 16), w2, b2.reshape(1, feat * _EMB),
      r_mat)


def _root_body(x_ref, root_ref, bias_ref, out_ref):
    out_ref[...] = lax.dot(
        x_ref[...], root_ref[...],
        preferred_element_type=jnp.float32) + bias_ref[...]


def _root_call(x, root, bias):
    feat = x.shape[1]
    tile = 1000
    full = lambda shape: pl.BlockSpec(shape, lambda i: (0, 0))
    return pl.pallas_call(
        _root_body,
        grid=(_N // tile,),
        in_specs=[
            pl.BlockSpec((tile, feat), lambda i: (i, 0)),
            full((feat, _EMB)),
            full((1, _EMB)),
        ],
        out_specs=pl.BlockSpec((tile, _EMB), lambda i: (i, 0)),
        out_shape=jax.ShapeDtypeStruct((_N, _EMB), jnp.float32),
    )(x, root, bias.reshape(1, _EMB))


def _update_body(p0_ref, p1_ref, x_ref, root_ref, bias_ref, out_ref):
    agg = p0_ref[...] + p1_ref[...]
    out_ref[...] = jnp.maximum(
        agg + lax.dot(x_ref[...], root_ref[...],
                      preferred_element_type=jnp.float32) + bias_ref[...], 0.0)


def _update_call(p0, p1, x, root, bias):
    feat = x.shape[1]
    tile = 1000
    full = lambda shape: pl.BlockSpec(shape, lambda i: (0, 0))
    return pl.pallas_call(
        _update_body,
        grid=(_N // tile,),
        in_specs=[
            pl.BlockSpec((tile, _EMB), lambda i: (i, 0)),
            pl.BlockSpec((tile, _EMB), lambda i: (i, 0)),
            pl.BlockSpec((tile, feat), lambda i: (i, 0)),
            full((feat, _EMB)),
            full((1, _EMB)),
        ],
        out_specs=pl.BlockSpec((tile, _EMB), lambda i: (i, 0)),
        out_shape=jax.ShapeDtypeStruct((_N, _EMB), jnp.float32),
    )(p0, p1, x, root, bias.reshape(1, _EMB))


def _final_body(p0_ref, p1_ref, x_ref, root_ref, bias_ref, batch_ref,
                l0w_ref, l0b_ref, l1w_ref, l1b_ref, out_ref, pool_ref):
    x2 = jnp.maximum(
        p0_ref[...] + p1_ref[...]
        + lax.dot(x_ref[...], root_ref[...],
                  preferred_element_type=jnp.float32) + bias_ref[...], 0.0)
    batch = batch_ref[...]

    def body(g, carry):
        m = jnp.where(batch == g, x2, -jnp.inf)
        pool_ref[pl.ds(g, 1), :] = jnp.max(m, axis=0, keepdims=True)
        return carry

    lax.fori_loop(0, _NG, body, 0)
    hidden = lax.dot(pool_ref[...], l0w_ref[...],
                     preferred_element_type=jnp.float32) + l0b_ref[...]
    out_ref[...] = lax.dot(hidden, l1w_ref[...],
                           preferred_element_type=jnp.float32) + l1b_ref[...]


def _final_call(p0, p1, x1, root, bias, batch, l0w, l0b, l1w, l1b):
    return pl.pallas_call(
        _final_body,
        out_shape=jax.ShapeDtypeStruct((_NG, 1), jnp.float32),
        scratch_shapes=[pltpu.VMEM((_NG, _EMB), jnp.float32)],
    )(p0, p1, x1, root, bias.reshape(1, _EMB), batch.reshape(_N, 1),
      l0w, l0b.reshape(1, _EMB), l1w, l1b.reshape(1, 1))


def _pad_idx(idx):
    return jnp.concatenate(
        [idx, jnp.zeros((_E_PAD - _E,), jnp.int32)]).reshape(_NCHP, _CHUNK)


@jax.jit
def kernel(x_p, x_d, edge_attr_p, edge_attr_d, edge_index_p, x_p_batch,
           nn0_W1, nn0_b1, nn0_W2, nn0_b2,
           nn1_W1, nn1_b1, nn1_W2, nn1_b2,
           root0, bias0, root1, bias1,
           lin0_W, lin0_b, lin1_W, lin1_b):
    src2d = _pad_idx(edge_index_p[0])
    dst2d = _pad_idx(edge_index_p[1])
    ea_pad = jnp.concatenate(
        [edge_attr_p, jnp.zeros((_E_PAD - _E, 16), jnp.float32)])

    zeros_n = jnp.zeros((_N, _EMB), jnp.float32)

    def bcast_mat(feat):
        cols = jnp.arange(feat * _EMB) // _EMB
        return (cols[None, :] == jnp.arange(feat)[:, None]).astype(jnp.float32)

    r32 = bcast_mat(_F_IN)
    r16 = bcast_mat(_EMB)

    scatter_add = _make_scatter_add()

    r0 = _root_call(x_p, root0, bias0)
    xg0 = _make_gather(_F_IN)(x_p, src2d)
    msg0 = _msg_call(ea_pad, xg0, nn0_W1, nn0_b1, nn0_W2, nn0_b2, r32)
    xg1, x1 = _make_fused_mid()(msg0, dst2d, src2d, r0, zeros_n)

    msg1 = _msg_call(ea_pad, xg1, nn1_W1, nn1_b1, nn1_W2, nn1_b2, r16)
    parts1 = scatter_add(msg1, dst2d, zeros_n)

    return _final_call(parts1[0], parts1[1], x1, root1, bias1, x_p_batch,
                       lin0_W, lin0_b, lin1_W, lin1_b)
